# Initial kernel scaffold; baseline (speedup 1.0000x reference)
import jax
import jax.numpy as jnp
from jax.experimental import pallas as pl


def kernel(feature, sp_embeddings, edge_index, edge_order, linear, attn_W, mlp_W, mlp_b, bn_gamma, bn_beta):
    raise NotImplementedError("write your pallas kernel here")


# speedup vs baseline: 5.3526x; 5.3526x over previous
import functools

import jax
import jax.numpy as jnp
from jax import lax
from jax.experimental import pallas as pl
from jax.experimental.pallas import tpu as pltpu
from jax.experimental.pallas import tpu_sc as plsc


def _tc1_body(f_ref, wb_ref, l0_ref, l1_ref, q_ref, m0_ref, m1_ref):
    f = f_ref[...]
    q_ref[...] = lax.dot_general(f, wb_ref[...], (((1,), (1,)), ((), ())),
                                 preferred_element_type=jnp.float32,
                                 precision=lax.Precision.HIGHEST)
    m0_ref[...] = jnp.dot(f, l0_ref[...], preferred_element_type=jnp.float32)
    m1_ref[...] = jnp.dot(f, l1_ref[...], preferred_element_type=jnp.float32)


def _tc1(feature, wb, linear, n_blk, blk):
    n, din = feature.shape
    emb = wb.shape[0]
    dout = linear.shape[2]
    return pl.pallas_call(
        _tc1_body,
        grid=(n_blk,),
        in_specs=[
            pl.BlockSpec((blk, din), lambda i: (i, 0)),
            pl.BlockSpec((emb, din), lambda i: (0, 0)),
            pl.BlockSpec((din, dout), lambda i: (0, 0)),
            pl.BlockSpec((din, dout), lambda i: (0, 0)),
        ],
        out_specs=[
            pl.BlockSpec((blk, emb), lambda i: (i, 0)),
            pl.BlockSpec((blk, dout), lambda i: (i, 0)),
            pl.BlockSpec((blk, dout), lambda i: (i, 0)),
        ],
        out_shape=[
            jax.ShapeDtypeStruct((n, emb), jnp.float32),
            jax.ShapeDtypeStruct((n, dout), jnp.float32),
            jax.ShapeDtypeStruct((n, dout), jnp.float32),
        ],
    )(feature, wb, linear[0], linear[1])


def _tc2_body(gus_ref, gud_ref, gq_ref, e_ref, d_ref, p4_ref):
    hi = lax.Precision.HIGHEST
    diff = gus_ref[...] - gud_ref[...]
    diffb = diff.astype(jnp.bfloat16).astype(jnp.float32)
    ones_e = jnp.ones((1, 16), jnp.float32)
    e = lax.dot_general(ones_e, diffb * gq_ref[...], (((1,), (1,)), ((), ())),
                        preferred_element_type=jnp.float32, precision=hi)
    s = lax.dot_general(ones_e, diff * diff, (((1,), (1,)), ((), ())),
                        preferred_element_type=jnp.float32, precision=hi)
    e_ref[0] = e
    d_ref[0] = 1.0 / (s + 1.0)
    p4_ref[0] = jnp.exp(0.25 * e)


def _tc2(gus, gud, gq, n_blk, blk):
    emb = gus.shape[1]
    return pl.pallas_call(
        _tc2_body,
        grid=(n_blk,),
        in_specs=[
            pl.BlockSpec((blk, emb), lambda i: (i, 0)),
            pl.BlockSpec((blk, emb), lambda i: (i, 0)),
            pl.BlockSpec((blk, emb), lambda i: (i, 0)),
        ],
        out_specs=[pl.BlockSpec((1, 1, blk), lambda i: (i, 0, 0))] * 3,
        out_shape=[jax.ShapeDtypeStruct((n_blk, 1, blk), jnp.float32)] * 3,
    )(gus, gud, gq)


def _tc3_body(den_ref, b_ref):
    x = den_ref[0] + den_ref[1]
    b_ref[...] = 4.0 * jnp.log(x)


def _tc3(den3):
    _, r, c = den3.shape
    return pl.pallas_call(
        _tc3_body,
        out_shape=jax.ShapeDtypeStruct((r, c), jnp.float32),
    )(den3)


def _tc4_body(a0_ref, a1_ref, dn_ref, f_ref, w_ref, bv_ref, g_ref, bt_ref,
              out_ref):
    h = a0_ref[...] + a1_ref[...]
    den = dn_ref[:, 0:1] + dn_ref[:, 1:2]
    mask = den > 0.0
    dsafe = jnp.where(mask, den, 1.0)
    h_agg = jnp.where(mask, h / dsafe, f_ref[...])
    x = jnp.dot(h_agg, w_ref[...], preferred_element_type=jnp.float32)
    x = jnp.maximum(x + bv_ref[...], 0.0)
    mean = jnp.mean(x, axis=0, keepdims=True)
    xc = x - mean
    var = jnp.mean(xc * xc, axis=0, keepdims=True)
    out_ref[...] = g_ref[...] * xc * lax.rsqrt(var + 1e-5) + bt_ref[...]


def _tc4(acc0, acc1, den_t, feature, mlp_W, mlp_b, bn_gamma, bn_beta):
    n, dout = feature.shape
    return pl.pallas_call(
        _tc4_body,
        out_shape=jax.ShapeDtypeStruct((n, dout), jnp.float32),
    )(acc0, acc1, den_t, feature, mlp_W, mlp_b.reshape(1, dout),
      bn_gamma.reshape(1, dout), bn_beta.reshape(1, dout))


_NC = 2
_NS = 16

_SC_PARAMS = pltpu.CompilerParams(use_tc_tiling_on_sc=False,
                                  needs_layout_passes=False)


def _worker_blocks(c, s, total_blocks):
    w = s * _NC + c
    base = total_blocks // (_NC * _NS)
    rem = total_blocks % (_NC * _NS)
    nb = base + jnp.where(w < rem, 1, 0)
    start = base * w + jnp.minimum(w, rem)
    return w, nb, start


def _node_chunks(n):
    ch = ((n + _NS - 1) // _NS + 7) // 8 * 8
    last = n - (_NS - 1) * ch
    assert last > 0 and last % 8 == 0
    return ch, last


def _sc_gather(q, u, src, dst):
    n, emb = q.shape
    e = src.shape[0]
    total_blocks = e // 128
    mesh = plsc.VectorSubcoreMesh(core_axis_name="c", subcore_axis_name="s")

    @functools.partial(
        pl.kernel,
        out_type=[jax.ShapeDtypeStruct((e, emb), jnp.float32)] * 3,
        mesh=mesh,
        compiler_params=_SC_PARAMS,
        scratch_types=[
            pltpu.VMEM((128,), jnp.int32),
            pltpu.VMEM((128,), jnp.int32),
            pltpu.VMEM((128, emb), jnp.float32),
            pltpu.VMEM((128, emb), jnp.float32),
            pltpu.VMEM((128, emb), jnp.float32),
            pltpu.SemaphoreType.DMA,
        ],
    )
    def k(q_hbm, u_hbm, src_hbm, dst_hbm, gq_hbm, gus_hbm, gud_hbm,
          src_v, dst_v, bq, bus, bud, sem):
        c = lax.axis_index("c")
        s = lax.axis_index("s")
        _, nb, start = _worker_blocks(c, s, total_blocks)

        def block(i, carry):
            base = (start + i) * 128
            pltpu.sync_copy(src_hbm.at[pl.ds(base, 128)], src_v)
            pltpu.sync_copy(dst_hbm.at[pl.ds(base, 128)], dst_v)
            pltpu.async_copy(q_hbm.at[src_v], bq, sem).wait()
            pltpu.async_copy(u_hbm.at[src_v], bus, sem).wait()
            pltpu.async_copy(u_hbm.at[dst_v], bud, sem).wait()
            pltpu.sync_copy(bq, gq_hbm.at[pl.ds(base, 128)])
            pltpu.sync_copy(bus, gus_hbm.at[pl.ds(base, 128)])
            pltpu.sync_copy(bud, gud_hbm.at[pl.ds(base, 128)])
            return carry

        lax.fori_loop(0, nb, block, 0)

    return k(q, u, src, dst)


def _sc_scatter_p4(p4, dst, n):
    e = p4.shape[0]
    total_blocks = e // 128
    ch, last = _node_chunks(n)
    mesh = plsc.VectorSubcoreMesh(core_axis_name="c", subcore_axis_name="s")

    @functools.partial(
        pl.kernel,
        out_type=jax.ShapeDtypeStruct((_NC, n), jnp.float32),
        mesh=mesh,
        compiler_params=_SC_PARAMS,
        scratch_types=[
            pltpu.VMEM((128,), jnp.int32),
            pltpu.VMEM((128,), jnp.float32),
            pltpu.VMEM(((ch + 15) // 16 * 16,), jnp.float32),
            pltpu.VMEM_SHARED((n,), jnp.float32),
        ],
    )
    def k(p4_hbm, dst_hbm, out_hbm, dst_v, p4_v, ztile, dtab):
        c = lax.axis_index("c")
        s = lax.axis_index("s")
        _, nb, start = _worker_blocks(c, s, total_blocks)

        zero = jnp.zeros((16,), jnp.float32)
        for t in range((ch + 15) // 16):
            ztile[pl.ds(t * 16, 16)] = zero

        @pl.when(s < _NS - 1)
        def _():
            pltpu.sync_copy(ztile.at[pl.ds(0, ch)], dtab.at[pl.ds(s * ch, ch)])

        @pl.when(s == _NS - 1)
        def _():
            pltpu.sync_copy(ztile.at[pl.ds(0, last)],
                            dtab.at[pl.ds((_NS - 1) * ch, last)])

        plsc.subcore_barrier()

        def block(i, carry):
            base = (start + i) * 128
            pltpu.sync_copy(dst_hbm.at[pl.ds(base, 128)], dst_v)
            pltpu.sync_copy(p4_hbm.at[pl.ds(base, 128)], p4_v)
            pltpu.sync_copy(p4_v, dtab.at[dst_v], add=True)
            return carry

        lax.fori_loop(0, nb, block, 0)
        plsc.subcore_barrier()

        @pl.when(s < _NS - 1)
        def _():
            pltpu.sync_copy(dtab.at[pl.ds(s * ch, ch)],
                            out_hbm.at[c, pl.ds(s * ch, ch)])

        @pl.when(s == _NS - 1)
        def _():
            pltpu.sync_copy(dtab.at[pl.ds((_NS - 1) * ch, last)],
                            out_hbm.at[c, pl.ds((_NS - 1) * ch, last)])

    return k(p4, dst)


def _sc_message_pass(e_arr, d_arr, dst, midx, b, m_cat, zacc, zn):
    e = e_arr.shape[0]
    n = b.shape[0]
    total_blocks = e // 128
    ch, last = _node_chunks(n)
    mesh = plsc.VectorSubcoreMesh(core_axis_name="c", subcore_axis_name="s")

    @functools.partial(
        pl.kernel,
        out_type=[jax.ShapeDtypeStruct((_NC, n, 128), jnp.float32),
                  jax.ShapeDtypeStruct((_NC, n), jnp.float32)],
        mesh=mesh,
        compiler_params=_SC_PARAMS,
        scratch_types=[
            pltpu.VMEM((n,), jnp.float32),
            pltpu.VMEM((128,), jnp.int32),
            pltpu.VMEM((128,), jnp.int32),
            pltpu.VMEM((128,), jnp.float32),
            pltpu.VMEM((128,), jnp.float32),
            pltpu.VMEM((128,), jnp.float32),
            pltpu.VMEM((128, 128), jnp.float32),
            pltpu.VMEM_SHARED((n, 128), jnp.float32),
            pltpu.VMEM_SHARED((n,), jnp.float32),
            pltpu.SemaphoreType.DMA,
        ],
    )
    def k(e_hbm, d_hbm, dst_hbm, midx_hbm, b_hbm, mcat_hbm, z_hbm, zn_hbm,
          outh_hbm, outd_hbm,
          btab, dst_v, midx_v, e_v, d_v, ex_v, msg_v, acc, accd, sem):
        c = lax.axis_index("c")
        s = lax.axis_index("s")
        _, nb, start = _worker_blocks(c, s, total_blocks)

        @pl.when(s < _NS - 1)
        def _():
            pltpu.sync_copy(z_hbm.at[pl.ds(s * ch, ch)],
                            acc.at[pl.ds(s * ch, ch)])
            pltpu.sync_copy(zn_hbm.at[pl.ds(s * ch, ch)],
                            accd.at[pl.ds(s * ch, ch)])

        @pl.when(s == _NS - 1)
        def _():
            pltpu.sync_copy(z_hbm.at[pl.ds((_NS - 1) * ch, last)],
                            acc.at[pl.ds((_NS - 1) * ch, last)])
            pltpu.sync_copy(zn_hbm.at[pl.ds((_NS - 1) * ch, last)],
                            accd.at[pl.ds((_NS - 1) * ch, last)])

        pltpu.sync_copy(b_hbm, btab)
        plsc.subcore_barrier()

        def block(i, carry):
            base = (start + i) * 128
            pltpu.sync_copy(dst_hbm.at[pl.ds(base, 128)], dst_v)
            pltpu.sync_copy(midx_hbm.at[pl.ds(base, 128)], midx_v)
            pltpu.sync_copy(e_hbm.at[pl.ds(base, 128)], e_v)
            pltpu.sync_copy(d_hbm.at[pl.ds(base, 128)], d_v)
            pltpu.async_copy(mcat_hbm.at[midx_v], msg_v, sem).wait()

            def group(g, carry2):
                dst16 = dst_v[pl.ds(g * 16, 16)]
                bb = plsc.load_gather(btab, [dst16])
                ex = jnp.exp(e_v[pl.ds(g * 16, 16)] - bb)
                ex_v[pl.ds(g * 16, 16)] = ex
                wv = ex * d_v[pl.ds(g * 16, 16)]
                for j in range(16):
                    ro = g * 16 + j
                    wj = jnp.full((16,), wv[j], jnp.float32)
                    for r in range(8):
                        msg_v[ro, pl.ds(r * 16, 16)] = (
                            msg_v[ro, pl.ds(r * 16, 16)] * wj)
                return carry2

            lax.fori_loop(0, 8, group, 0)
            pltpu.sync_copy(msg_v, acc.at[dst_v], add=True)
            pltpu.sync_copy(ex_v, accd.at[dst_v], add=True)
            return carry

        lax.fori_loop(0, nb, block, 0)
        plsc.subcore_barrier()

        @pl.when(s < _NS - 1)
        def _():
            pltpu.sync_copy(acc.at[pl.ds(s * ch, ch)],
                            outh_hbm.at[c, pl.ds(s * ch, ch)])
            pltpu.sync_copy(accd.at[pl.ds(s * ch, ch)],
                            outd_hbm.at[c, pl.ds(s * ch, ch)])

        @pl.when(s == _NS - 1)
        def _():
            pltpu.sync_copy(acc.at[pl.ds((_NS - 1) * ch, last)],
                            outh_hbm.at[c, pl.ds((_NS - 1) * ch, last)])
            pltpu.sync_copy(accd.at[pl.ds((_NS - 1) * ch, last)],
                            outd_hbm.at[c, pl.ds((_NS - 1) * ch, last)])

    return k(e_arr, d_arr, dst, midx, b, m_cat, zacc, zn)


def kernel(feature, sp_embeddings, edge_index, edge_order, linear, attn_W,
           mlp_W, mlp_b, bn_gamma, bn_beta):
    n, din = feature.shape
    e = edge_index.shape[1]
    src = edge_index[0]
    dst = edge_index[1]

    wb = attn_W.astype(jnp.bfloat16).astype(jnp.float32)
    q, m0, m1 = _tc1(feature, wb, linear, n_blk=10, blk=n // 10)
    m_cat = jnp.concatenate([m0, m1], axis=0)
    midx = src + n * edge_order

    gq, gus, gud = _sc_gather(q, sp_embeddings, src, dst)

    n_blk2 = 80
    e_blks, d_blks, p4_blks = _tc2(gus, gud, gq,
                                   n_blk=n_blk2, blk=e // n_blk2)
    e_arr = e_blks.reshape(e)
    d_arr = d_blks.reshape(e)
    p4 = p4_blks.reshape(e)

    den4 = _sc_scatter_p4(p4, dst, n)
    b = _tc3(den4.reshape(_NC, n // 125, 125)).reshape(n)

    zacc = jnp.zeros((n, 128), jnp.float32)
    zn = jnp.zeros((n,), jnp.float32)
    acc, accd = _sc_message_pass(e_arr, d_arr, dst, midx, b, m_cat, zacc, zn)

    return _tc4(acc[0], acc[1], accd.T, feature, mlp_W, mlp_b,
                bn_gamma, bn_beta)


# speedup vs baseline: 9.1164x; 1.7032x over previous
import functools

import jax
import jax.numpy as jnp
from jax import lax
from jax.experimental import pallas as pl
from jax.experimental.pallas import tpu as pltpu
from jax.experimental.pallas import tpu_sc as plsc


def _tc1_body(f_ref, wb_ref, l0_ref, l1_ref, q_ref, m0_ref, m1_ref):
    f = f_ref[...]
    q_ref[...] = lax.dot_general(f, wb_ref[...], (((1,), (1,)), ((), ())),
                                 preferred_element_type=jnp.float32,
                                 precision=lax.Precision.HIGHEST)
    m0_ref[...] = jnp.dot(f, l0_ref[...], preferred_element_type=jnp.float32)
    m1_ref[...] = jnp.dot(f, l1_ref[...], preferred_element_type=jnp.float32)


def _tc1(feature, wb, linear, n_blk, blk):
    n, din = feature.shape
    emb = wb.shape[0]
    dout = linear.shape[2]
    return pl.pallas_call(
        _tc1_body,
        grid=(n_blk,),
        in_specs=[
            pl.BlockSpec((blk, din), lambda i: (i, 0)),
            pl.BlockSpec((emb, din), lambda i: (0, 0)),
            pl.BlockSpec((din, dout), lambda i: (0, 0)),
            pl.BlockSpec((din, dout), lambda i: (0, 0)),
        ],
        out_specs=[
            pl.BlockSpec((blk, emb), lambda i: (i, 0)),
            pl.BlockSpec((blk, dout), lambda i: (i, 0)),
            pl.BlockSpec((blk, dout), lambda i: (i, 0)),
        ],
        out_shape=[
            jax.ShapeDtypeStruct((n, emb), jnp.float32),
            jax.ShapeDtypeStruct((n, dout), jnp.float32),
            jax.ShapeDtypeStruct((n, dout), jnp.float32),
        ],
    )(feature, wb, linear[0], linear[1])


def _tc3_body(den_ref, b_ref):
    x = den_ref[0] + den_ref[1]
    b_ref[...] = 4.0 * jnp.log(x)


def _tc3(den3):
    _, r, c = den3.shape
    return pl.pallas_call(
        _tc3_body,
        out_shape=jax.ShapeDtypeStruct((r, c), jnp.float32),
    )(den3)


def _tc4_body(a0_ref, a1_ref, dn_ref, f_ref, w_ref, bv_ref, g_ref, bt_ref,
              out_ref):
    h = a0_ref[...] + a1_ref[...]
    den = dn_ref[:, 0:1] + dn_ref[:, 1:2]
    mask = den > 0.0
    dsafe = jnp.where(mask, den, 1.0)
    h_agg = jnp.where(mask, h / dsafe, f_ref[...])
    x = jnp.dot(h_agg, w_ref[...], preferred_element_type=jnp.float32)
    x = jnp.maximum(x + bv_ref[...], 0.0)
    mean = jnp.mean(x, axis=0, keepdims=True)
    xc = x - mean
    var = jnp.mean(xc * xc, axis=0, keepdims=True)
    out_ref[...] = g_ref[...] * xc * lax.rsqrt(var + 1e-5) + bt_ref[...]


def _tc4(acc0, acc1, den_t, feature, mlp_W, mlp_b, bn_gamma, bn_beta):
    n, dout = feature.shape
    return pl.pallas_call(
        _tc4_body,
        out_shape=jax.ShapeDtypeStruct((n, dout), jnp.float32),
    )(acc0, acc1, den_t, feature, mlp_W, mlp_b.reshape(1, dout),
      bn_gamma.reshape(1, dout), bn_beta.reshape(1, dout))


_NC = 2
_NS = 16

_SC_PARAMS = pltpu.CompilerParams(use_tc_tiling_on_sc=False,
                                  needs_layout_passes=False)


def _worker_blocks(c, s, total_blocks):
    w = s * _NC + c
    base = total_blocks // (_NC * _NS)
    rem = total_blocks % (_NC * _NS)
    nb = base + jnp.where(w < rem, 1, 0)
    start = base * w + jnp.minimum(w, rem)
    return w, nb, start


def _node_chunks(n):
    ch = ((n + _NS - 1) // _NS + 7) // 8 * 8
    last = n - (_NS - 1) * ch
    assert last > 0 and last % 8 == 0
    return ch, last


def _sc_edge_pass(q, u, src, dst, n):
    emb = q.shape[1]
    e = src.shape[0]
    total_blocks = e // 128
    ch, last = _node_chunks(n)
    mesh = plsc.VectorSubcoreMesh(core_axis_name="c", subcore_axis_name="s")

    @functools.partial(
        pl.kernel,
        out_type=[jax.ShapeDtypeStruct((e,), jnp.float32),
                  jax.ShapeDtypeStruct((e,), jnp.float32),
                  jax.ShapeDtypeStruct((_NC, n), jnp.float32)],
        mesh=mesh,
        compiler_params=_SC_PARAMS,
        scratch_types=[
            pltpu.VMEM((128,), jnp.int32),
            pltpu.VMEM((128,), jnp.int32),
            pltpu.VMEM((128, emb), jnp.float32),
            pltpu.VMEM((128, emb), jnp.float32),
            pltpu.VMEM((128, emb), jnp.float32),
            pltpu.VMEM((128,), jnp.float32),
            pltpu.VMEM((128,), jnp.float32),
            pltpu.VMEM((128,), jnp.float32),
            pltpu.VMEM(((ch + 15) // 16 * 16,), jnp.float32),
            pltpu.VMEM_SHARED((n,), jnp.float32),
            pltpu.SemaphoreType.DMA,
        ],
    )
    def k(q_hbm, u_hbm, src_hbm, dst_hbm, e_hbm, d_hbm, den_hbm,
          src_v, dst_v, bq, bus, bud, estg, dstg, pstg, ztile, dtab, sem):
        c = lax.axis_index("c")
        s = lax.axis_index("s")
        _, nb, start = _worker_blocks(c, s, total_blocks)

        zero = jnp.zeros((16,), jnp.float32)
        for t in range((ch + 15) // 16):
            ztile[pl.ds(t * 16, 16)] = zero

        @pl.when(s < _NS - 1)
        def _():
            pltpu.sync_copy(ztile.at[pl.ds(0, ch)], dtab.at[pl.ds(s * ch, ch)])

        @pl.when(s == _NS - 1)
        def _():
            pltpu.sync_copy(ztile.at[pl.ds(0, last)],
                            dtab.at[pl.ds((_NS - 1) * ch, last)])

        plsc.subcore_barrier()

        iota16 = lax.broadcasted_iota(jnp.int32, (16,), 0)

        def block(i, carry):
            base = (start + i) * 128
            pltpu.sync_copy(src_hbm.at[pl.ds(base, 128)], src_v)
            pltpu.sync_copy(dst_hbm.at[pl.ds(base, 128)], dst_v)
            pltpu.async_copy(q_hbm.at[src_v], bq, sem).wait()
            pltpu.async_copy(u_hbm.at[src_v], bus, sem).wait()
            pltpu.async_copy(u_hbm.at[dst_v], bud, sem).wait()

            def group(g, carry2):
                rows = g * 16 + iota16
                acc_e = jnp.zeros((16,), jnp.float32)
                acc_s = jnp.zeros((16,), jnp.float32)
                for dim in range(emb):
                    col = jnp.full((16,), dim, jnp.int32)
                    qv = plsc.load_gather(bq, [rows, col])
                    usv = plsc.load_gather(bus, [rows, col])
                    udv = plsc.load_gather(bud, [rows, col])
                    diffv = usv - udv
                    bits = plsc.bitcast(diffv, jnp.int32)
                    r = bits + jnp.int32(0x7FFF) + ((bits >> 16) & 1)
                    diffb = plsc.bitcast(r & jnp.int32(-65536), jnp.float32)
                    acc_e = acc_e + diffb * qv
                    acc_s = acc_s + diffv * diffv
                estg[pl.ds(g * 16, 16)] = acc_e
                dstg[pl.ds(g * 16, 16)] = 1.0 / (acc_s + 1.0)
                pstg[pl.ds(g * 16, 16)] = jnp.exp(0.25 * acc_e)
                return carry2

            lax.fori_loop(0, 8, group, 0)
            pltpu.sync_copy(estg, e_hbm.at[pl.ds(base, 128)])
            pltpu.sync_copy(dstg, d_hbm.at[pl.ds(base, 128)])
            pltpu.sync_copy(pstg, dtab.at[dst_v], add=True)
            return carry

        lax.fori_loop(0, nb, block, 0)
        plsc.subcore_barrier()

        @pl.when(s < _NS - 1)
        def _():
            pltpu.sync_copy(dtab.at[pl.ds(s * ch, ch)],
                            den_hbm.at[c, pl.ds(s * ch, ch)])

        @pl.when(s == _NS - 1)
        def _():
            pltpu.sync_copy(dtab.at[pl.ds((_NS - 1) * ch, last)],
                            den_hbm.at[c, pl.ds((_NS - 1) * ch, last)])

    return k(q, u, src, dst)


def _sc_message_pass(e_arr, d_arr, dst, midx, b, m_cat, zacc, zn):
    e = e_arr.shape[0]
    n = b.shape[0]
    total_blocks = e // 128
    ch, last = _node_chunks(n)
    mesh = plsc.VectorSubcoreMesh(core_axis_name="c", subcore_axis_name="s")

    @functools.partial(
        pl.kernel,
        out_type=[jax.ShapeDtypeStruct((_NC, n, 128), jnp.float32),
                  jax.ShapeDtypeStruct((_NC, n), jnp.float32)],
        mesh=mesh,
        compiler_params=_SC_PARAMS,
        scratch_types=[
            pltpu.VMEM((n,), jnp.float32),
            pltpu.VMEM((128,), jnp.int32),
            pltpu.VMEM((128,), jnp.int32),
            pltpu.VMEM((128,), jnp.float32),
            pltpu.VMEM((128,), jnp.float32),
            pltpu.VMEM((128,), jnp.float32),
            pltpu.VMEM((128, 128), jnp.float32),
            pltpu.VMEM_SHARED((n, 128), jnp.float32),
            pltpu.VMEM_SHARED((n,), jnp.float32),
            pltpu.SemaphoreType.DMA,
        ],
    )
    def k(e_hbm, d_hbm, dst_hbm, midx_hbm, b_hbm, mcat_hbm, z_hbm, zn_hbm,
          outh_hbm, outd_hbm,
          btab, dst_v, midx_v, e_v, d_v, ex_v, msg_v, acc, accd, sem):
        c = lax.axis_index("c")
        s = lax.axis_index("s")
        _, nb, start = _worker_blocks(c, s, total_blocks)

        @pl.when(s < _NS - 1)
        def _():
            pltpu.sync_copy(z_hbm.at[pl.ds(s * ch, ch)],
                            acc.at[pl.ds(s * ch, ch)])
            pltpu.sync_copy(zn_hbm.at[pl.ds(s * ch, ch)],
                            accd.at[pl.ds(s * ch, ch)])

        @pl.when(s == _NS - 1)
        def _():
            pltpu.sync_copy(z_hbm.at[pl.ds((_NS - 1) * ch, last)],
                            acc.at[pl.ds((_NS - 1) * ch, last)])
            pltpu.sync_copy(zn_hbm.at[pl.ds((_NS - 1) * ch, last)],
                            accd.at[pl.ds((_NS - 1) * ch, last)])

        pltpu.sync_copy(b_hbm, btab)
        plsc.subcore_barrier()

        def block(i, carry):
            base = (start + i) * 128
            pltpu.sync_copy(dst_hbm.at[pl.ds(base, 128)], dst_v)
            pltpu.sync_copy(midx_hbm.at[pl.ds(base, 128)], midx_v)
            pltpu.sync_copy(e_hbm.at[pl.ds(base, 128)], e_v)
            pltpu.sync_copy(d_hbm.at[pl.ds(base, 128)], d_v)
            pltpu.async_copy(mcat_hbm.at[midx_v], msg_v, sem).wait()

            def group(g, carry2):
                dst16 = dst_v[pl.ds(g * 16, 16)]
                bb = plsc.load_gather(btab, [dst16])
                ex = jnp.exp(e_v[pl.ds(g * 16, 16)] - bb)
                ex_v[pl.ds(g * 16, 16)] = ex
                wv = ex * d_v[pl.ds(g * 16, 16)]
                for j in range(16):
                    ro = g * 16 + j
                    wj = jnp.full((16,), wv[j], jnp.float32)
                    for r in range(8):
                        msg_v[ro, pl.ds(r * 16, 16)] = (
                            msg_v[ro, pl.ds(r * 16, 16)] * wj)
                return carry2

            lax.fori_loop(0, 8, group, 0)
            pltpu.sync_copy(msg_v, acc.at[dst_v], add=True)
            pltpu.sync_copy(ex_v, accd.at[dst_v], add=True)
            return carry

        lax.fori_loop(0, nb, block, 0)
        plsc.subcore_barrier()

        @pl.when(s < _NS - 1)
        def _():
            pltpu.sync_copy(acc.at[pl.ds(s * ch, ch)],
                            outh_hbm.at[c, pl.ds(s * ch, ch)])
            pltpu.sync_copy(accd.at[pl.ds(s * ch, ch)],
                            outd_hbm.at[c, pl.ds(s * ch, ch)])

        @pl.when(s == _NS - 1)
        def _():
            pltpu.sync_copy(acc.at[pl.ds((_NS - 1) * ch, last)],
                            outh_hbm.at[c, pl.ds((_NS - 1) * ch, last)])
            pltpu.sync_copy(accd.at[pl.ds((_NS - 1) * ch, last)],
                            outd_hbm.at[c, pl.ds((_NS - 1) * ch, last)])

    return k(e_arr, d_arr, dst, midx, b, m_cat, zacc, zn)


def kernel(feature, sp_embeddings, edge_index, edge_order, linear, attn_W,
           mlp_W, mlp_b, bn_gamma, bn_beta):
    n, din = feature.shape
    e = edge_index.shape[1]
    src = edge_index[0]
    dst = edge_index[1]

    wb = attn_W.astype(jnp.bfloat16).astype(jnp.float32)
    q, m0, m1 = _tc1(feature, wb, linear, n_blk=10, blk=n // 10)
    m_cat = jnp.concatenate([m0, m1], axis=0)
    midx = src + n * edge_order

    e_arr, d_arr, den4 = _sc_edge_pass(q, sp_embeddings, src, dst, n)

    b = _tc3(den4.reshape(_NC, n // 125, 125)).reshape(n)

    zacc = jnp.zeros((n, 128), jnp.float32)
    zn = jnp.zeros((n,), jnp.float32)
    acc, accd = _sc_message_pass(e_arr, d_arr, dst, midx, b, m_cat, zacc, zn)

    return _tc4(acc[0], acc[1], accd.T, feature, mlp_W, mlp_b,
                bn_gamma, bn_beta)
